# baseline (speedup 1.0000x reference)
import functools

import jax
import jax.numpy as jnp
from jax import lax
from jax.experimental import pallas as pl
from jax.experimental.pallas import tpu as pltpu
from jax.experimental.pallas import tpu_sc as plsc

_DELTA_V = 0.5
_DELTA_D = 1.5
_ALPHA = 1.0
_BETA = 1.0
_GAMMA = 0.001
_K = 64
_L = 16
_NC = 2
_NS = 16
_B = 8
_N = 32768
_RPC = _B // _NC
_PTS = _N // _NS
_GRP = _PTS // _L
_RW = 2 * _K * _L
_CNT0 = _K * _L
_ACCW = _RPC * _RW
_SLICE = _ACCW // _NS


def _rsqrt(x):
    i = plsc.bitcast(x, jnp.int32)
    i = 0x5F3759DF - (i >> 1)
    y = plsc.bitcast(i, jnp.float32)
    for _ in range(3):
        y = y * (1.5 - 0.5 * x * y * y)
    return y


def _sqrt(x):
    return x * _rsqrt(jnp.maximum(x, 1e-30))


def _sc_body(e_hbm, ids_hbm, out_hbm,
             ebuf, idbuf, accbuf, sumbuf, meansbuf, invbuf, redbuf, redout,
             outbuf, shared_all, shared_red):
    c = lax.axis_index("c")
    s = lax.axis_index("s")
    wid = c * _NS + s
    lane = lax.iota(jnp.int32, _L)
    zeros16 = jnp.zeros((_L,), jnp.float32)

    def zero_acc(i, _):
        accbuf[pl.ds(i * _L, _L)] = zeros16
        return 0
    lax.fori_loop(0, _ACCW // _L, zero_acc, 0)

    def pa_row(r, _):
        row = c * _RPC + r
        pltpu.sync_copy(
            e_hbm.at[pl.ds((row * _N + s * _PTS) * _L, _PTS * _L)], ebuf)
        pltpu.sync_copy(
            ids_hbm.at[pl.ds(row * _N + s * _PTS, _PTS)],
            idbuf.at[pl.ds(r * _PTS, _PTS)])
        abase = r * _RW
        rid0 = idbuf[pl.ds(r * _PTS, _L)][0]

        def pa(g, st):
            racc, rcntv, rid = st
            base = g * _L
            idv = idbuf[pl.ds(r * _PTS + base, _L)]
            uni = jnp.logical_and(idv[0] == rid, idv[_L - 1] == rid)

            def fast(racc, rcntv):
                t = [ebuf[pl.ds((base + j) * _L, _L)] for j in range(_L)]
                while len(t) > 1:
                    t = [t[i] + t[i + 1] for i in range(0, len(t), 2)]
                return racc + t[0], rcntv + 16.0, rid

            def slow(racc, rcntv):
                off = abase + rid * _L
                accbuf[pl.ds(off, _L)] = accbuf[pl.ds(off, _L)] + racc
                offc = off + _CNT0
                accbuf[pl.ds(offc, _L)] = accbuf[pl.ds(offc, _L)] + rcntv
                for j in range(_L):
                    o = abase + idv[j] * _L
                    accbuf[pl.ds(o, _L)] = (
                        accbuf[pl.ds(o, _L)]
                        + ebuf[pl.ds((base + j) * _L, _L)])
                    oc = o + _CNT0
                    accbuf[pl.ds(oc, _L)] = accbuf[pl.ds(oc, _L)] + 1.0
                return zeros16, zeros16, idv[_L - 1]

            return lax.cond(uni, fast, slow, racc, rcntv)

        racc, rcntv, rid = lax.fori_loop(
            0, _GRP, pa, (zeros16, zeros16, rid0))
        off = abase + rid * _L
        accbuf[pl.ds(off, _L)] = accbuf[pl.ds(off, _L)] + racc
        offc = off + _CNT0
        accbuf[pl.ds(offc, _L)] = accbuf[pl.ds(offc, _L)] + rcntv
        return 0
    lax.fori_loop(0, _RPC, pa_row, 0)

    pltpu.sync_copy(accbuf, shared_all.at[s])
    plsc.subcore_barrier()
    pltpu.sync_copy(shared_all.at[:, pl.ds(s * _SLICE, _SLICE)], redbuf)
    for cc in range(_SLICE // _L):
        tot = redbuf[0, pl.ds(cc * _L, _L)]
        for t in range(1, _NS):
            tot = tot + redbuf[t, pl.ds(cc * _L, _L)]
        redout[pl.ds(cc * _L, _L)] = tot
    pltpu.sync_copy(redout, shared_red.at[pl.ds(s * _SLICE, _SLICE)])
    plsc.subcore_barrier()
    pltpu.sync_copy(shared_red, sumbuf)

    def mk(i, _):
        r = i // _K
        k = i % _K
        sbase = r * _RW + k * _L
        mbase = r * _K * _L + k * _L
        cntv = sumbuf[pl.ds(sbase + _CNT0, _L)]
        ivv = 1.0 / jnp.maximum(cntv, 1.0)
        invbuf[pl.ds(mbase, _L)] = ivv
        meansbuf[pl.ds(mbase, _L)] = sumbuf[pl.ds(sbase, _L)] * ivv
        return 0
    lax.fori_loop(0, _RPC * _K, mk, 0)

    def pb_row(r, carry):
        vacc, pairacc, regacc = carry
        row = c * _RPC + r
        mbase0 = r * _K * _L
        pltpu.sync_copy(
            e_hbm.at[pl.ds((row * _N + s * _PTS) * _L, _PTS * _L)], ebuf)
        rid0 = idbuf[pl.ds(r * _PTS, _L)][0]

        def pb(g, st):
            va, mcur, icur, rid = st
            base = g * _L
            idv = idbuf[pl.ds(r * _PTS + base, _L)]
            uni = jnp.logical_and(idv[0] == rid, idv[_L - 1] == rid)

            def fast(va, mcur, icur):
                ssqv = zeros16
                for j in range(_L):
                    d = ebuf[pl.ds((base + j) * _L, _L)] - mcur
                    ssqv = jnp.where(lane == j, jnp.sum(d * d), ssqv)
                distv = _sqrt(ssqv + 1e-12)
                hv = jnp.maximum(distv - _DELTA_V, 0.0)
                return va + hv * hv * icur, mcur, icur, rid

            def slow(va, mcur, icur):
                ssqv = zeros16
                invgv = zeros16
                for j in range(_L):
                    idj = idv[j]
                    d = (ebuf[pl.ds((base + j) * _L, _L)]
                         - meansbuf[pl.ds(mbase0 + idj * _L, _L)])
                    ssqv = jnp.where(lane == j, jnp.sum(d * d), ssqv)
                    invgv = jnp.where(
                        lane == j, invbuf[pl.ds(mbase0 + idj * _L, _L)],
                        invgv)
                distv = _sqrt(ssqv + 1e-12)
                hv = jnp.maximum(distv - _DELTA_V, 0.0)
                nid = idv[_L - 1]
                return (va + hv * hv * invgv,
                        meansbuf[pl.ds(mbase0 + nid * _L, _L)],
                        invbuf[pl.ds(mbase0 + nid * _L, _L)], nid)

            return lax.cond(uni, fast, slow, va, mcur, icur)

        vacc, _, _, _ = lax.fori_loop(
            0, _GRP, pb,
            (vacc, meansbuf[pl.ds(mbase0 + rid0 * _L, _L)],
             invbuf[pl.ds(mbase0 + rid0 * _L, _L)], rid0))

        def pk(kk, pa_acc):
            k = s * 4 + kk
            mk_v = meansbuf[pl.ds(mbase0 + k * _L, _L)]

            def pjg(jg, acc):
                sqv = jnp.ones((_L,), jnp.float32)
                for j in range(_L):
                    mj = meansbuf[pl.ds(mbase0 + (jg * _L + j) * _L, _L)]
                    d = mk_v - mj
                    sqv = jnp.where(lane == j, jnp.sum(d * d), sqv)
                pd = _sqrt(sqv)
                hp = jnp.maximum(2.0 * _DELTA_D - pd, 0.0)
                jidx = jg * _L + lane
                return acc + jnp.where(jidx == k, 0.0, hp * hp)
            return lax.fori_loop(0, _K // _L, pjg, pa_acc)
        pairacc = lax.fori_loop(0, 4, pk, pairacc)

        nsqv = jnp.ones((_L,), jnp.float32)
        for kk in range(4):
            mk_v = meansbuf[pl.ds(mbase0 + (s * 4 + kk) * _L, _L)]
            nsqv = jnp.where(lane == kk, jnp.sum(mk_v * mk_v), nsqv)
        regacc = regacc + jnp.where(lane < 4, _sqrt(nsqv + 1e-12), 0.0)

        return (vacc, pairacc, regacc)

    init = (jnp.zeros((_L,), jnp.float32),) * 3
    vacc, pairacc, regacc = lax.fori_loop(0, _RPC, pb_row, init)

    varp = jnp.sum(vacc)
    distp = jnp.sum(pairacc)
    regp = jnp.sum(regacc)
    outv = jnp.where(lane == 0, varp,
                     jnp.where(lane == 1, distp,
                               jnp.where(lane == 2, regp, 0.0)))
    outbuf[...] = outv
    pltpu.sync_copy(outbuf, out_hbm.at[wid])


@functools.partial(
    pl.kernel,
    mesh=plsc.VectorSubcoreMesh(core_axis_name="c", subcore_axis_name="s"),
    compiler_params=pltpu.CompilerParams(
        needs_layout_passes=False, use_tc_tiling_on_sc=False),
    out_type=jax.ShapeDtypeStruct((_NC * _NS, _L), jnp.float32),
    scratch_types=[
        pltpu.VMEM((_PTS * _L,), jnp.float32),
        pltpu.VMEM((_RPC * _PTS,), jnp.int32),
        pltpu.VMEM((_ACCW,), jnp.float32),
        pltpu.VMEM((_ACCW,), jnp.float32),
        pltpu.VMEM((_RPC * _K * _L,), jnp.float32),
        pltpu.VMEM((_RPC * _K * _L,), jnp.float32),
        pltpu.VMEM((_NS, _SLICE), jnp.float32),
        pltpu.VMEM((_SLICE,), jnp.float32),
        pltpu.VMEM((_L,), jnp.float32),
        pltpu.VMEM_SHARED((_NS, _ACCW), jnp.float32),
        pltpu.VMEM_SHARED((_ACCW,), jnp.float32),
    ],
)
def _sc_kernel(e_hbm, ids_hbm, out_hbm, *scratch):
    _sc_body(e_hbm, ids_hbm, out_hbm, *scratch)


@jax.jit
def kernel(embeddings, instance_ids):
    e_flat = embeddings.reshape(-1)
    ids_flat = instance_ids.astype(jnp.int32).reshape(-1)
    p = _sc_kernel(e_flat, ids_flat)
    num_pairs = _K * (_K - 1) / 2.0
    var_loss = jnp.sum(p[:, 0]) / (_K * _B)
    dist_loss = jnp.sum(p[:, 1]) / (2.0 * num_pairs * _B)
    reg_loss = jnp.sum(p[:, 2]) / (_K * _B)
    total = _ALPHA * var_loss + _BETA * dist_loss + _GAMMA * reg_loss
    return (total, var_loss, dist_loss, reg_loss)


# speedup vs baseline: 2.0326x; 2.0326x over previous
import functools

import jax
import jax.numpy as jnp
from jax import lax
from jax.experimental import pallas as pl
from jax.experimental.pallas import tpu as pltpu
from jax.experimental.pallas import tpu_sc as plsc

_DELTA_V = 0.5
_DELTA_D = 1.5
_ALPHA = 1.0
_BETA = 1.0
_GAMMA = 0.001
_K = 64
_B = 8
_N = 32768
_D = 16


_TC_ROWS = 6
_CHUNK = 8192


def _dot(a, b, dims):
    return jax.lax.dot_general(
        a, b, (dims, ((), ())), preferred_element_type=jnp.float32)


def _tc_row_body(ids_ref, et_ref, out_ref, oh_ref):
    n = et_ref.shape[2]
    d = et_ref.shape[1]
    nch = n // _CHUNK

    iota_k = jax.lax.broadcasted_iota(jnp.int32, (_K, _CHUNK), 0)
    ones_row = jnp.ones((1, _CHUNK), jnp.float32)
    sums_t = jnp.zeros((d, _K), jnp.float32)
    counts = jnp.zeros((1, _K), jnp.float32)
    for c in range(nch):
        sl = slice(c * _CHUNK, (c + 1) * _CHUNK)
        idc = ids_ref[0, :, sl]
        ohc = jnp.where(idc == iota_k, 1.0, 0.0)
        oh_ref[:, sl] = ohc
        etc = et_ref[0, :, sl]
        sums_t = sums_t + _dot(etc, ohc, ((1,), (1,)))
        counts = counts + _dot(ones_row, ohc, ((1,), (1,)))
    cnt = jnp.maximum(counts, 1.0)
    inv = 1.0 / cnt
    means_t = sums_t * inv

    var_acc = jnp.zeros((), jnp.float32)
    for c in range(nch):
        sl = slice(c * _CHUNK, (c + 1) * _CHUNK)
        ohc = oh_ref[:, sl]
        mg_t = _dot(means_t, ohc, ((1,), (0,)))
        invg = _dot(inv, ohc, ((1,), (0,)))
        etc = et_ref[0, :, sl]
        diff = etc - mg_t
        ssq = jnp.sum(diff * diff, axis=0) + 1e-12
        dist = jnp.sqrt(ssq)
        h = jnp.maximum(dist - _DELTA_V, 0.0)
        var_acc = var_acc + jnp.sum(h * h * invg[0])
    var_loss = var_acc / _K

    md = means_t[:, :, None] - means_t[:, None, :]
    sq = jnp.sum(md * md, axis=0)
    ii = jax.lax.broadcasted_iota(jnp.int32, (_K, _K), 0)
    jj = jax.lax.broadcasted_iota(jnp.int32, (_K, _K), 1)
    offdiag = ii != jj
    pd = jnp.sqrt(jnp.where(offdiag, sq, 1.0))
    hp = jnp.maximum(2.0 * _DELTA_D - pd, 0.0)
    num_pairs = _K * (_K - 1) / 2.0
    dist_loss = jnp.sum(jnp.where(offdiag, hp * hp, 0.0)) / (2.0 * num_pairs)

    reg_loss = jnp.mean(jnp.sqrt(jnp.sum(means_t * means_t, axis=0) + 1e-12))

    lane = jax.lax.broadcasted_iota(jnp.int32, (1, 128), 1)
    vec = jnp.where(lane == 0, var_loss,
                    jnp.where(lane == 1, dist_loss,
                              jnp.where(lane == 2, reg_loss, 0.0)))
    out_ref[0] = vec


def _tc_kernel(emb_t, ids3):
    return pl.pallas_call(
        _tc_row_body,
        grid=(_TC_ROWS,),
        in_specs=[
            pl.BlockSpec((1, 1, _N), lambda r: (r, 0, 0)),
            pl.BlockSpec((1, _D, _N), lambda r: (r, 0, 0)),
        ],
        out_specs=pl.BlockSpec((1, 1, 128), lambda r: (r, 0, 0)),
        out_shape=jax.ShapeDtypeStruct((_TC_ROWS, 1, 128), jnp.float32),
        scratch_shapes=[pltpu.VMEM((_K, _N), jnp.float32)],
    )(ids3, emb_t)


_L = 16
_NC = 2
_NS = 16
_SC_ROWS = 2
_RPC = _SC_ROWS // _NC
_PTS = _N // _NS
_GRP = _PTS // _L
_RW = 2 * _K * _L
_CNT0 = _K * _L
_ACCW = _RPC * _RW
_SLICE = _ACCW // _NS


def _rsqrt(x):
    i = plsc.bitcast(x, jnp.int32)
    i = 0x5F3759DF - (i >> 1)
    y = plsc.bitcast(i, jnp.float32)
    for _ in range(3):
        y = y * (1.5 - 0.5 * x * y * y)
    return y


def _sqrt(x):
    return x * _rsqrt(jnp.maximum(x, 1e-30))


def _sc_body(e_hbm, ids_hbm, out_hbm,
             ebuf, idbuf, accbuf, sumbuf, meansbuf, invbuf, redbuf, redout,
             outbuf, shared_all, shared_red):
    c = lax.axis_index("c")
    s = lax.axis_index("s")
    wid = c * _NS + s
    lane = lax.iota(jnp.int32, _L)
    zeros16 = jnp.zeros((_L,), jnp.float32)

    def zero_acc(i, _):
        accbuf[pl.ds(i * _L, _L)] = zeros16
        return 0
    lax.fori_loop(0, _ACCW // _L, zero_acc, 0)

    def pa_row(r, _):
        row = c * _RPC + r
        pltpu.sync_copy(
            e_hbm.at[pl.ds((row * _N + s * _PTS) * _L, _PTS * _L)], ebuf)
        pltpu.sync_copy(
            ids_hbm.at[pl.ds(row * _N + s * _PTS, _PTS)],
            idbuf.at[pl.ds(r * _PTS, _PTS)])
        abase = r * _RW
        rid0 = idbuf[pl.ds(r * _PTS, _L)][0]

        def pa(g, st):
            racc, rcntv, rid = st
            base = g * _L
            idv = idbuf[pl.ds(r * _PTS + base, _L)]
            uni = jnp.logical_and(idv[0] == rid, idv[_L - 1] == rid)

            def fast(racc, rcntv):
                t = [ebuf[pl.ds((base + j) * _L, _L)] for j in range(_L)]
                while len(t) > 1:
                    t = [t[i] + t[i + 1] for i in range(0, len(t), 2)]
                return racc + t[0], rcntv + 16.0, rid

            def slow(racc, rcntv):
                off = abase + rid * _L
                accbuf[pl.ds(off, _L)] = accbuf[pl.ds(off, _L)] + racc
                offc = off + _CNT0
                accbuf[pl.ds(offc, _L)] = accbuf[pl.ds(offc, _L)] + rcntv
                for j in range(_L):
                    o = abase + idv[j] * _L
                    accbuf[pl.ds(o, _L)] = (
                        accbuf[pl.ds(o, _L)]
                        + ebuf[pl.ds((base + j) * _L, _L)])
                    oc = o + _CNT0
                    accbuf[pl.ds(oc, _L)] = accbuf[pl.ds(oc, _L)] + 1.0
                return zeros16, zeros16, idv[_L - 1]

            return lax.cond(uni, fast, slow, racc, rcntv)

        racc, rcntv, rid = lax.fori_loop(
            0, _GRP, pa, (zeros16, zeros16, rid0))
        off = abase + rid * _L
        accbuf[pl.ds(off, _L)] = accbuf[pl.ds(off, _L)] + racc
        offc = off + _CNT0
        accbuf[pl.ds(offc, _L)] = accbuf[pl.ds(offc, _L)] + rcntv
        return 0
    lax.fori_loop(0, _RPC, pa_row, 0)

    pltpu.sync_copy(accbuf, shared_all.at[s])
    plsc.subcore_barrier()
    pltpu.sync_copy(shared_all.at[:, pl.ds(s * _SLICE, _SLICE)], redbuf)
    for cc in range(_SLICE // _L):
        tot = redbuf[0, pl.ds(cc * _L, _L)]
        for t in range(1, _NS):
            tot = tot + redbuf[t, pl.ds(cc * _L, _L)]
        redout[pl.ds(cc * _L, _L)] = tot
    pltpu.sync_copy(redout, shared_red.at[pl.ds(s * _SLICE, _SLICE)])
    plsc.subcore_barrier()
    pltpu.sync_copy(shared_red, sumbuf)

    def mk(i, _):
        r = i // _K
        k = i % _K
        sbase = r * _RW + k * _L
        mbase = r * _K * _L + k * _L
        cntv = sumbuf[pl.ds(sbase + _CNT0, _L)]
        ivv = 1.0 / jnp.maximum(cntv, 1.0)
        invbuf[pl.ds(mbase, _L)] = ivv
        meansbuf[pl.ds(mbase, _L)] = sumbuf[pl.ds(sbase, _L)] * ivv
        return 0
    lax.fori_loop(0, _RPC * _K, mk, 0)

    def pb_row(r, carry):
        vacc, pairacc, regacc = carry
        row = c * _RPC + r
        mbase0 = r * _K * _L
        pltpu.sync_copy(
            e_hbm.at[pl.ds((row * _N + s * _PTS) * _L, _PTS * _L)], ebuf)
        rid0 = idbuf[pl.ds(r * _PTS, _L)][0]

        def pb(g, st):
            va, mcur, icur, rid = st
            base = g * _L
            idv = idbuf[pl.ds(r * _PTS + base, _L)]
            uni = jnp.logical_and(idv[0] == rid, idv[_L - 1] == rid)

            def fast(va, mcur, icur):
                ssqv = zeros16
                for j in range(_L):
                    d = ebuf[pl.ds((base + j) * _L, _L)] - mcur
                    ssqv = jnp.where(lane == j, jnp.sum(d * d), ssqv)
                distv = _sqrt(ssqv + 1e-12)
                hv = jnp.maximum(distv - _DELTA_V, 0.0)
                return va + hv * hv * icur, mcur, icur, rid

            def slow(va, mcur, icur):
                ssqv = zeros16
                invgv = zeros16
                for j in range(_L):
                    idj = idv[j]
                    d = (ebuf[pl.ds((base + j) * _L, _L)]
                         - meansbuf[pl.ds(mbase0 + idj * _L, _L)])
                    ssqv = jnp.where(lane == j, jnp.sum(d * d), ssqv)
                    invgv = jnp.where(
                        lane == j, invbuf[pl.ds(mbase0 + idj * _L, _L)],
                        invgv)
                distv = _sqrt(ssqv + 1e-12)
                hv = jnp.maximum(distv - _DELTA_V, 0.0)
                nid = idv[_L - 1]
                return (va + hv * hv * invgv,
                        meansbuf[pl.ds(mbase0 + nid * _L, _L)],
                        invbuf[pl.ds(mbase0 + nid * _L, _L)], nid)

            return lax.cond(uni, fast, slow, va, mcur, icur)

        vacc, _, _, _ = lax.fori_loop(
            0, _GRP, pb,
            (vacc, meansbuf[pl.ds(mbase0 + rid0 * _L, _L)],
             invbuf[pl.ds(mbase0 + rid0 * _L, _L)], rid0))

        def pk(kk, pa_acc):
            k = s * 4 + kk
            mk_v = meansbuf[pl.ds(mbase0 + k * _L, _L)]

            def pjg(jg, acc):
                sqv = jnp.ones((_L,), jnp.float32)
                for j in range(_L):
                    mj = meansbuf[pl.ds(mbase0 + (jg * _L + j) * _L, _L)]
                    d = mk_v - mj
                    sqv = jnp.where(lane == j, jnp.sum(d * d), sqv)
                pd = _sqrt(sqv)
                hp = jnp.maximum(2.0 * _DELTA_D - pd, 0.0)
                jidx = jg * _L + lane
                return acc + jnp.where(jidx == k, 0.0, hp * hp)
            return lax.fori_loop(0, _K // _L, pjg, pa_acc)
        pairacc = lax.fori_loop(0, 4, pk, pairacc)

        nsqv = jnp.ones((_L,), jnp.float32)
        for kk in range(4):
            mk_v = meansbuf[pl.ds(mbase0 + (s * 4 + kk) * _L, _L)]
            nsqv = jnp.where(lane == kk, jnp.sum(mk_v * mk_v), nsqv)
        regacc = regacc + jnp.where(lane < 4, _sqrt(nsqv + 1e-12), 0.0)

        return (vacc, pairacc, regacc)

    init = (jnp.zeros((_L,), jnp.float32),) * 3
    vacc, pairacc, regacc = lax.fori_loop(0, _RPC, pb_row, init)

    varp = jnp.sum(vacc)
    distp = jnp.sum(pairacc)
    regp = jnp.sum(regacc)
    outv = jnp.where(lane == 0, varp,
                     jnp.where(lane == 1, distp,
                               jnp.where(lane == 2, regp, 0.0)))
    outbuf[...] = outv
    pltpu.sync_copy(outbuf, out_hbm.at[wid])


@functools.partial(
    pl.kernel,
    mesh=plsc.VectorSubcoreMesh(core_axis_name="c", subcore_axis_name="s"),
    compiler_params=pltpu.CompilerParams(
        needs_layout_passes=False, use_tc_tiling_on_sc=False),
    out_type=jax.ShapeDtypeStruct((_NC * _NS, _L), jnp.float32),
    scratch_types=[
        pltpu.VMEM((_PTS * _L,), jnp.float32),
        pltpu.VMEM((_RPC * _PTS,), jnp.int32),
        pltpu.VMEM((_ACCW,), jnp.float32),
        pltpu.VMEM((_ACCW,), jnp.float32),
        pltpu.VMEM((_RPC * _K * _L,), jnp.float32),
        pltpu.VMEM((_RPC * _K * _L,), jnp.float32),
        pltpu.VMEM((_NS, _SLICE), jnp.float32),
        pltpu.VMEM((_SLICE,), jnp.float32),
        pltpu.VMEM((_L,), jnp.float32),
        pltpu.VMEM_SHARED((_NS, _ACCW), jnp.float32),
        pltpu.VMEM_SHARED((_ACCW,), jnp.float32),
    ],
)
def _sc_kernel(e_hbm, ids_hbm, out_hbm, *scratch):
    _sc_body(e_hbm, ids_hbm, out_hbm, *scratch)


@jax.jit
def kernel(embeddings, instance_ids):
    ids = instance_ids.astype(jnp.int32)
    num_pairs = _K * (_K - 1) / 2.0

    e_sc = embeddings[_TC_ROWS:].reshape(-1)
    ids_sc = ids[_TC_ROWS:].reshape(-1)
    p = _sc_kernel(e_sc, ids_sc)

    emb_t = embeddings[:_TC_ROWS].transpose(0, 2, 1)
    ids3 = ids[:_TC_ROWS].reshape(_TC_ROWS, 1, _N)
    out_tc = _tc_kernel(emb_t, ids3)

    var_loss = (jnp.sum(out_tc[:, 0, 0]) + jnp.sum(p[:, 0]) / _K) / _B
    dist_loss = (jnp.sum(out_tc[:, 0, 1])
                 + jnp.sum(p[:, 1]) / (2.0 * num_pairs)) / _B
    reg_loss = (jnp.sum(out_tc[:, 0, 2]) + jnp.sum(p[:, 2]) / _K) / _B
    total = _ALPHA * var_loss + _BETA * dist_loss + _GAMMA * reg_loss
    return (total, var_loss, dist_loss, reg_loss)


# speedup vs baseline: 4.7974x; 2.3603x over previous
import functools

import jax
import jax.numpy as jnp
from jax.experimental import pallas as pl
from jax.experimental.pallas import tpu as pltpu

_DELTA_V = 0.5
_DELTA_D = 1.5
_ALPHA = 1.0
_BETA = 1.0
_GAMMA = 0.001
_K = 64

_CHUNK = 8192


def _dot(a, b, dims):
    return jax.lax.dot_general(
        a, b, (dims, ((), ())), preferred_element_type=jnp.float32)


def _row_body(ids_ref, et_ref, out_ref, oh_ref):
    B = et_ref.shape[2]
    D = et_ref.shape[1]
    nch = B // _CHUNK

    iota_k = jax.lax.broadcasted_iota(jnp.int32, (_K, _CHUNK), 0)
    ones_row = jnp.ones((1, _CHUNK), jnp.float32)
    sums_t = jnp.zeros((D, _K), jnp.float32)
    counts = jnp.zeros((1, _K), jnp.float32)
    for c in range(nch):
        sl = slice(c * _CHUNK, (c + 1) * _CHUNK)
        idc = ids_ref[0, :, sl]
        ohc = jnp.where(idc == iota_k, 1.0, 0.0)
        oh_ref[:, sl] = ohc
        etc = et_ref[0, :, sl]
        sums_t = sums_t + _dot(etc, ohc, ((1,), (1,)))
        counts = counts + _dot(ones_row, ohc, ((1,), (1,)))
    cnt = jnp.maximum(counts, 1.0)
    inv = 1.0 / cnt
    means_t = sums_t * inv

    var_acc = jnp.zeros((), jnp.float32)
    for c in range(nch):
        sl = slice(c * _CHUNK, (c + 1) * _CHUNK)
        ohc = oh_ref[:, sl]
        mg_t = _dot(means_t, ohc, ((1,), (0,)))
        invg = _dot(inv, ohc, ((1,), (0,)))
        etc = et_ref[0, :, sl]
        diff = etc - mg_t
        ssq = jnp.sum(diff * diff, axis=0) + 1e-12
        dist = jnp.sqrt(ssq)
        h = jnp.maximum(dist - _DELTA_V, 0.0)
        var_acc = var_acc + jnp.sum(h * h * invg[0])
    var_loss = var_acc / _K

    md = means_t[:, :, None] - means_t[:, None, :]
    sq = jnp.sum(md * md, axis=0)
    ii = jax.lax.broadcasted_iota(jnp.int32, (_K, _K), 0)
    jj = jax.lax.broadcasted_iota(jnp.int32, (_K, _K), 1)
    offdiag = ii != jj
    pd = jnp.sqrt(jnp.where(offdiag, sq, 1.0))
    hp = jnp.maximum(2.0 * _DELTA_D - pd, 0.0)
    num_pairs = _K * (_K - 1) / 2.0
    dist_loss = jnp.sum(jnp.where(offdiag, hp * hp, 0.0)) / (2.0 * num_pairs)

    reg_loss = jnp.mean(jnp.sqrt(jnp.sum(means_t * means_t, axis=0) + 1e-12))

    lane = jax.lax.broadcasted_iota(jnp.int32, (1, 128), 1)
    vec = jnp.where(lane == 0, var_loss,
                    jnp.where(lane == 1, dist_loss,
                              jnp.where(lane == 2, reg_loss, 0.0)))
    out_ref[0] = vec


@functools.partial(jax.jit, static_argnames=())
def kernel(embeddings, instance_ids):
    Bt, N, D = embeddings.shape
    ids3 = instance_ids.reshape(Bt, 1, N).astype(jnp.int32)
    emb_t = embeddings.transpose(0, 2, 1)
    out = pl.pallas_call(
        _row_body,
        grid=(Bt,),
        in_specs=[
            pl.BlockSpec((1, 1, N), lambda r: (r, 0, 0)),
            pl.BlockSpec((1, D, N), lambda r: (r, 0, 0)),
        ],
        out_specs=pl.BlockSpec((1, 1, 128), lambda r: (r, 0, 0)),
        out_shape=jax.ShapeDtypeStruct((Bt, 1, 128), jnp.float32),
        scratch_shapes=[pltpu.VMEM((_K, N), jnp.float32)],
    )(ids3, emb_t)
    var_loss = jnp.mean(out[:, 0, 0])
    dist_loss = jnp.mean(out[:, 0, 1])
    reg_loss = jnp.mean(out[:, 0, 2])
    total = _ALPHA * var_loss + _BETA * dist_loss + _GAMMA * reg_loss
    return (total, var_loss, dist_loss, reg_loss)


# speedup vs baseline: 4.8107x; 1.0028x over previous
import functools

import jax
import jax.numpy as jnp
from jax.experimental import pallas as pl
from jax.experimental.pallas import tpu as pltpu

_DELTA_V = 0.5
_DELTA_D = 1.5
_ALPHA = 1.0
_BETA = 1.0
_GAMMA = 0.001
_K = 64

_CHUNK = 16384


def _dot(a, b, dims):
    return jax.lax.dot_general(
        a, b, (dims, ((), ())), preferred_element_type=jnp.float32)


def _row_body(ids_ref, et_ref, out_ref, oh_ref):
    B = et_ref.shape[2]
    D = et_ref.shape[1]
    nch = B // _CHUNK

    iota_k = jax.lax.broadcasted_iota(jnp.int32, (_K, _CHUNK), 0)
    ones_row = jnp.ones((1, _CHUNK), jnp.float32)
    sums_t = jnp.zeros((D, _K), jnp.float32)
    counts = jnp.zeros((1, _K), jnp.float32)
    for c in range(nch):
        sl = slice(c * _CHUNK, (c + 1) * _CHUNK)
        idc = ids_ref[0, :, sl]
        ohc = jnp.where(idc == iota_k, 1.0, 0.0)
        oh_ref[:, sl] = ohc
        etc = et_ref[0, :, sl]
        sums_t = sums_t + _dot(etc, ohc, ((1,), (1,)))
        counts = counts + _dot(ones_row, ohc, ((1,), (1,)))
    cnt = jnp.maximum(counts, 1.0)
    inv = 1.0 / cnt
    means_t = sums_t * inv

    var_acc = jnp.zeros((), jnp.float32)
    for c in range(nch):
        sl = slice(c * _CHUNK, (c + 1) * _CHUNK)
        ohc = oh_ref[:, sl]
        mg_t = _dot(means_t, ohc, ((1,), (0,)))
        invg = _dot(inv, ohc, ((1,), (0,)))
        etc = et_ref[0, :, sl]
        diff = etc - mg_t
        ssq = jnp.sum(diff * diff, axis=0) + 1e-12
        dist = jnp.sqrt(ssq)
        h = jnp.maximum(dist - _DELTA_V, 0.0)
        var_acc = var_acc + jnp.sum(h * h * invg[0])
    var_loss = var_acc / _K

    md = means_t[:, :, None] - means_t[:, None, :]
    sq = jnp.sum(md * md, axis=0)
    ii = jax.lax.broadcasted_iota(jnp.int32, (_K, _K), 0)
    jj = jax.lax.broadcasted_iota(jnp.int32, (_K, _K), 1)
    offdiag = ii != jj
    pd = jnp.sqrt(jnp.where(offdiag, sq, 1.0))
    hp = jnp.maximum(2.0 * _DELTA_D - pd, 0.0)
    num_pairs = _K * (_K - 1) / 2.0
    dist_loss = jnp.sum(jnp.where(offdiag, hp * hp, 0.0)) / (2.0 * num_pairs)

    reg_loss = jnp.mean(jnp.sqrt(jnp.sum(means_t * means_t, axis=0) + 1e-12))

    lane = jax.lax.broadcasted_iota(jnp.int32, (1, 128), 1)
    vec = jnp.where(lane == 0, var_loss,
                    jnp.where(lane == 1, dist_loss,
                              jnp.where(lane == 2, reg_loss, 0.0)))
    out_ref[0] = vec


@functools.partial(jax.jit, static_argnames=())
def kernel(embeddings, instance_ids):
    Bt, N, D = embeddings.shape
    ids3 = instance_ids.reshape(Bt, 1, N).astype(jnp.int32)
    emb_t = embeddings.transpose(0, 2, 1)
    out = pl.pallas_call(
        _row_body,
        grid=(Bt,),
        in_specs=[
            pl.BlockSpec((1, 1, N), lambda r: (r, 0, 0)),
            pl.BlockSpec((1, D, N), lambda r: (r, 0, 0)),
        ],
        out_specs=pl.BlockSpec((1, 1, 128), lambda r: (r, 0, 0)),
        out_shape=jax.ShapeDtypeStruct((Bt, 1, 128), jnp.float32),
        scratch_shapes=[pltpu.VMEM((_K, N), jnp.float32)],
    )(ids3, emb_t)
    var_loss = jnp.mean(out[:, 0, 0])
    dist_loss = jnp.mean(out[:, 0, 1])
    reg_loss = jnp.mean(out[:, 0, 2])
    total = _ALPHA * var_loss + _BETA * dist_loss + _GAMMA * reg_loss
    return (total, var_loss, dist_loss, reg_loss)
